# baseline (speedup 1.0000x reference)
import functools

import jax
import jax.numpy as jnp
from jax import lax
from jax.experimental import pallas as pl
from jax.experimental.pallas import tpu as pltpu
from jax.experimental.pallas import tpu_sc as plsc

B_ROWS = 16384
B_COLS = 26
D = 64
NB = B_ROWS * B_COLS
NC = 2
NS = 16
NW = NC * NS
PER_W = NB // NW
CHUNK = 128
NCHUNK = PER_W // CHUNK


def kernel(input_, weight):
    idx = input_.reshape(NW, NCHUNK, CHUNK)
    mesh = plsc.VectorSubcoreMesh(core_axis_name="c", subcore_axis_name="s")

    @functools.partial(
        pl.kernel,
        mesh=mesh,
        out_type=jax.ShapeDtypeStruct((NB, D), jnp.float32),
        scratch_types=[
            pltpu.VMEM((NCHUNK, CHUNK), jnp.int32),
            pltpu.VMEM((CHUNK, D), jnp.float32),
            pltpu.SemaphoreType.DMA,
        ],
        compiler_params=pltpu.CompilerParams(use_tc_tiling_on_sc=False),
    )
    def sc_gather(table_hbm, idx_hbm, out_hbm, idx_v, rows_v, sem):
        wid = lax.axis_index("s") * NC + lax.axis_index("c")
        base = wid * PER_W
        pltpu.sync_copy(idx_hbm.at[wid], idx_v)

        def body(j, carry):
            pltpu.async_copy(table_hbm.at[idx_v.at[j]], rows_v, sem).wait()
            pltpu.sync_copy(rows_v, out_hbm.at[pl.ds(base + j * CHUNK, CHUNK)])
            return carry

        lax.fori_loop(0, NCHUNK, body, 0)

    out = sc_gather(weight, idx)
    return out.reshape(B_ROWS, B_COLS, D)


# speedup vs baseline: 1.0735x; 1.0735x over previous
import functools

import jax
import jax.numpy as jnp
from jax import lax
from jax.experimental import pallas as pl
from jax.experimental.pallas import tpu as pltpu
from jax.experimental.pallas import tpu_sc as plsc

B_ROWS = 16384
B_COLS = 26
D = 64
NB = B_ROWS * B_COLS
NC = 2
NS = 16
NW = NC * NS
PER_W = NB // NW
CHUNK = 128
NCHUNK = PER_W // CHUNK
RC = 4
RROWS = RC * CHUNK
NR = NCHUNK // RC
ROW_BYTES = D * 4
OUT_BYTES = RROWS * ROW_BYTES


def kernel(input_, weight):
    idx = input_.reshape(NW, NCHUNK, CHUNK)
    mesh = plsc.VectorSubcoreMesh(core_axis_name="c", subcore_axis_name="s")

    @functools.partial(
        pl.kernel,
        mesh=mesh,
        out_type=jax.ShapeDtypeStruct((NB, D), jnp.float32),
        scratch_types=[
            pltpu.VMEM((NCHUNK, CHUNK), jnp.int32),
            pltpu.VMEM((RROWS, D), jnp.float32),
            pltpu.VMEM((RROWS, D), jnp.float32),
            pltpu.SemaphoreType.DMA,
            pltpu.SemaphoreType.DMA,
            pltpu.SemaphoreType.DMA,
            pltpu.SemaphoreType.DMA,
        ],
        compiler_params=pltpu.CompilerParams(use_tc_tiling_on_sc=False),
    )
    def sc_gather(table_hbm, idx_hbm, out_hbm, idx_v,
                  rows0, rows1, gsem0, gsem1, osem0, osem1):
        wid = lax.axis_index("s") * NC + lax.axis_index("c")
        base = wid * PER_W
        pltpu.sync_copy(idx_hbm.at[wid], idx_v)

        def fire_gathers(r, buf, gsem):
            for c in range(RC):
                pltpu.async_copy(
                    table_hbm.at[idx_v.at[r * RC + c]],
                    buf.at[pl.ds(c * CHUNK, CHUNK)],
                    gsem,
                )

        def wait_gathers(buf, gsem):
            for c in range(RC):
                pltpu.make_async_copy(
                    table_hbm.at[pl.ds(0, CHUNK)],
                    buf.at[pl.ds(c * CHUNK, CHUNK)],
                    gsem,
                ).wait()

        def out_slice(r):
            return out_hbm.at[pl.ds(base + r * RROWS, RROWS)]

        def half_round(r, bufA, gsemA, osemA, bufB, gsemB, osemB):
            wait_gathers(bufA, gsemA)
            pltpu.async_copy(bufA, out_slice(r), osemA)

            @pl.when(r >= 1)
            def _():
                pltpu.make_async_copy(bufB, out_slice(0), osemB).wait()

            @pl.when(r + 1 < NR)
            def _():
                fire_gathers(r + 1, bufB, gsemB)

        fire_gathers(0, rows0, gsem0)

        def body(k, carry):
            r = 2 * k
            half_round(r, rows0, gsem0, osem0, rows1, gsem1, osem1)
            half_round(r + 1, rows1, gsem1, osem1, rows0, gsem0, osem0)
            return carry

        lax.fori_loop(0, NR // 2, body, 0)
        pltpu.make_async_copy(rows1, out_slice(0), osem1).wait()

    out = sc_gather(weight, idx)
    return out.reshape(B_ROWS, B_COLS, D)
